# baseline (device time: 176546 ns/iter reference)
import jax
import jax.numpy as jnp
from jax import lax
from jax.experimental import pallas as pl
from jax.experimental.pallas import tpu as pltpu

N_DEV = 8
M_BLK = 512
K_SH = 512
N_OUT = 8192

PART_W = (2688, 2688, 2816)
PART_OFF = (0, 2688, 5376)
AXES = ((0, 1, 2), (1, 2, 0), (2, 0, 1))

_COORDS = {0: (0, 0, 0), 1: (1, 0, 0), 2: (1, 1, 0), 3: (0, 1, 0),
           4: (0, 0, 1), 5: (1, 0, 1), 6: (1, 1, 1), 7: (0, 1, 1)}
_C2I = {c: i for i, c in _COORDS.items()}


def _flip_table(mask: int) -> list[int]:
    out = []
    for i in range(N_DEV):
        cx, cy, cz = _COORDS[i]
        out.append(_C2I[(cx ^ (mask & 1), cy ^ ((mask >> 1) & 1),
                         cz ^ ((mask >> 2) & 1))])
    return out


FLIP = [_flip_table(d) for d in range(N_DEV)]
IDM = [FLIP[d][0] for d in range(N_DEV)]
for _d in range(N_DEV):
    assert all(FLIP[_d][_i] == _i ^ IDM[_d] for _i in range(N_DEV))

FP8 = jnp.float8_e4m3fn


def _body(x_ref, w_ref, sx_ref, sw_ref, out_ref,
          wb0, wb1, wb2, x_buf,
          w_send, w_recv, x_send, x_recv):
    me = lax.axis_index("i")
    wbufs = (wb0, wb1, wb2)
    nbr = [me ^ IDM[1 << a] for a in range(3)]

    def sem_idx(p, s, k):
        return p * 7 + (0, 1, 3)[s] + k

    def send_masks(p, s):
        e = [1 << a for a in AXES[p]]
        if s == 0:
            return [0]
        if s == 1:
            return [0, e[0]]
        return [0, e[0], e[1], e[0] ^ e[1]]

    def w_xchg(p, s, k, m):
        a = AXES[p][s]
        src = (w_ref.at[:, pl.ds(PART_OFF[p], PART_W[p])] if m == 0
               else wbufs[p].at[m - 1])
        return pltpu.make_async_remote_copy(
            src_ref=src,
            dst_ref=wbufs[p].at[(m ^ (1 << a)) - 1],
            send_sem=w_send.at[sem_idx(p, s, k)],
            recv_sem=w_recv.at[sem_idx(p, s, k)],
            device_id=(nbr[a],),
            device_id_type=pl.DeviceIdType.MESH,
        )

    barrier = pltpu.get_barrier_semaphore()
    for d in range(1, N_DEV):
        pl.semaphore_signal(
            barrier, inc=1,
            device_id=(me ^ IDM[d],),
            device_id_type=pl.DeviceIdType.MESH,
        )
    pl.semaphore_wait(barrier, N_DEV - 1)

    for p in range(3):
        w_xchg(p, 0, 0, 0).start()

    for d in range(1, N_DEV):
        dst = me ^ IDM[d]
        pltpu.make_async_remote_copy(
            src_ref=x_ref.at[pl.ds(dst * M_BLK, M_BLK), :],
            dst_ref=x_buf.at[d - 1],
            send_sem=x_send.at[d - 1],
            recv_sem=x_recv.at[d - 1],
            device_id=(dst,),
            device_id_type=pl.DeviceIdType.MESH,
        ).start()

    x_own = x_ref[pl.ds(me * M_BLK, M_BLK), :]
    for p in range(3):
        out_ref[:, PART_OFF[p]:PART_OFF[p] + PART_W[p]] = jnp.dot(
            x_own, w_ref[:, PART_OFF[p]:PART_OFF[p] + PART_W[p]],
            preferred_element_type=jnp.float32)

    x_waited: set[int] = set()
    for s in range(3):
        for p in range(3):
            a = AXES[p][s]
            masks = send_masks(p, s)
            for k, m in enumerate(masks):
                w_xchg(p, s, k, m).wait_recv()
            if s < 2:
                for k, m in enumerate(send_masks(p, s + 1)):
                    w_xchg(p, s + 1, k, m).start()
            for m in masks:
                d = m ^ (1 << a)
                if d not in x_waited:
                    x_waited.add(d)
                    pltpu.make_async_remote_copy(
                        src_ref=x_buf.at[d - 1],
                        dst_ref=x_buf.at[d - 1],
                        send_sem=x_send.at[d - 1],
                        recv_sem=x_recv.at[d - 1],
                        device_id=(nbr[a],),
                        device_id_type=pl.DeviceIdType.MESH,
                    ).wait_recv()
                out_ref[:, PART_OFF[p]:PART_OFF[p] + PART_W[p]] += jnp.dot(
                    x_buf[d - 1], wbufs[p][d - 1],
                    preferred_element_type=jnp.float32)

    scale = sx_ref[0] * sw_ref[0]
    for t in range(8):
        cols = pl.ds(t * (N_OUT // 8), N_OUT // 8)
        y = out_ref[:, cols] * scale
        out_ref[:, cols] = y * jax.nn.sigmoid(y)

    for s in range(3):
        for p in range(3):
            for k, m in enumerate(send_masks(p, s)):
                w_xchg(p, s, k, m).wait_send()
    for d in range(1, N_DEV):
        pltpu.make_async_remote_copy(
            src_ref=x_ref.at[pl.ds(0, M_BLK), :],
            dst_ref=x_buf.at[0],
            send_sem=x_send.at[d - 1],
            recv_sem=x_recv.at[0],
            device_id=(nbr[0],),
            device_id_type=pl.DeviceIdType.MESH,
        ).wait_send()


def kernel(x, w_mat, scale_x, scale_w):
    x8 = x.astype(FP8)
    w8 = w_mat.astype(FP8)
    return pl.pallas_call(
        _body,
        out_shape=jax.ShapeDtypeStruct((M_BLK, N_OUT), jnp.float32),
        in_specs=[
            pl.BlockSpec(memory_space=pltpu.VMEM),
            pl.BlockSpec(memory_space=pltpu.VMEM),
            pl.BlockSpec(memory_space=pltpu.SMEM),
            pl.BlockSpec(memory_space=pltpu.SMEM),
        ],
        out_specs=pl.BlockSpec(memory_space=pltpu.VMEM),
        scratch_shapes=[
            pltpu.VMEM((N_DEV - 1, K_SH, PART_W[0]), FP8),
            pltpu.VMEM((N_DEV - 1, K_SH, PART_W[1]), FP8),
            pltpu.VMEM((N_DEV - 1, K_SH, PART_W[2]), FP8),
            pltpu.VMEM((N_DEV - 1, M_BLK, K_SH), FP8),
            pltpu.SemaphoreType.DMA((21,)),
            pltpu.SemaphoreType.DMA((21,)),
            pltpu.SemaphoreType.DMA((N_DEV - 1,)),
            pltpu.SemaphoreType.DMA((N_DEV - 1,)),
        ],
        compiler_params=pltpu.CompilerParams(
            collective_id=0,
            vmem_limit_bytes=100 * 1024 * 1024,
        ),
    )(x8, w8, scale_x, scale_w)
